# baseline (speedup 1.0000x reference)
import functools

import jax
import jax.numpy as jnp
from jax import lax
from jax.experimental import pallas as pl
from jax.experimental.pallas import tpu as pltpu
from jax.experimental.pallas import tpu_sc as plsc

_K = 8192
_D = 64
_BETA = 0.25
_N = 16384
_MT = 1024
_KC = 1024
_NW = 32
_BPW = _N // _NW
_CH = 128

_PRECISION = lax.Precision.DEFAULT


def _proj_body(cb_ref, pw_ref, pb_ref, qc_ref, qsq_ref):
    qc = lax.dot_general(
        cb_ref[...], pw_ref[...], (((1,), (1,)), ((), ())),
        precision=_PRECISION, preferred_element_type=jnp.float32,
    ) + pb_ref[...]
    qc_ref[...] = qc
    qsq_ref[...] = lax.dot_general(
        jnp.ones((1, _D), jnp.float32), qc * qc, (((1,), (1,)), ((), ())),
        precision=lax.Precision.HIGHEST, preferred_element_type=jnp.float32)


def _project(codebook, proj_w, proj_b2d):
    return pl.pallas_call(
        _proj_body,
        out_shape=[
            jax.ShapeDtypeStruct((_K, _D), jnp.float32),
            jax.ShapeDtypeStruct((1, _K), jnp.float32),
        ],
    )(codebook, proj_w, proj_b2d)


def _argmin_body(z_ref, qc_ref, qsq_ref, idx_ref):
    zt = z_ref[...]
    z_sq = jnp.sum(zt * zt, axis=1, keepdims=True)
    zt2 = -2.0 * zt
    iota_col = lax.broadcasted_iota(jnp.int32, (_KC, 1), 0).astype(jnp.float32)
    iota2c = jnp.concatenate(
        [iota_col, iota_col * iota_col, jnp.ones((_KC, 1), jnp.float32)],
        axis=1)
    bv = jnp.full((_MT, 1), jnp.inf, jnp.float32)
    bi = jnp.zeros((_MT, 1), jnp.int32)
    for c in range(_K // _KC):
        qcc = qc_ref[pl.ds(c * _KC, _KC), :]
        qs = qsq_ref[:, pl.ds(c * _KC, _KC)]
        dots2 = lax.dot_general(
            zt2, qcc, (((1,), (1,)), ((), ())),
            precision=_PRECISION, preferred_element_type=jnp.float32)
        d = (z_sq + qs) + dots2
        cmin = jnp.min(d, axis=1, keepdims=True)
        eqf = jnp.where(d == cmin, 1.0, 0.0).astype(jnp.float32)
        mom = lax.dot_general(
            eqf, iota2c, (((1,), (0,)), ((), ())),
            precision=lax.Precision.HIGHEST, preferred_element_type=jnp.float32)
        s = mom[:, 0:1]
        cnt = mom[:, 2:3]
        r = jnp.sqrt(cnt * mom[:, 1:2] - s * s)
        cidx = ((s - r) / cnt).astype(jnp.int32)
        upd = cmin < bv
        bi = jnp.where(upd, cidx + (c * _KC), bi)
        bv = jnp.where(upd, cmin, bv)
    idx_ref[...] = bi


def _argmin(zf, qc, qsq):
    return pl.pallas_call(
        _argmin_body,
        grid=(_N // _MT,),
        in_specs=[
            pl.BlockSpec((_MT, _D), lambda i: (i, 0)),
            pl.BlockSpec((_K, _D), lambda i: (0, 0)),
            pl.BlockSpec((1, _K), lambda i: (0, 0)),
        ],
        out_specs=pl.BlockSpec((_MT, 1), lambda i: (i, 0)),
        out_shape=jax.ShapeDtypeStruct((_N, 1), jnp.int32),
    )(zf, qc, qsq)


def _sc_gather_count(qc, idx2d):
    mesh = plsc.VectorSubcoreMesh(core_axis_name="c", subcore_axis_name="s")
    nchunk = _BPW // _CH
    stripe = _K // 16

    @functools.partial(
        pl.kernel,
        mesh=mesh,
        compiler_params=pltpu.CompilerParams(use_tc_tiling_on_sc=False),
        out_type=[
            jax.ShapeDtypeStruct((_N, _D), jnp.float32),
            jax.ShapeDtypeStruct((2, _K, 16), jnp.float32),
        ],
        scratch_types=[
            pltpu.VMEM((nchunk, _CH), jnp.int32),
            pltpu.VMEM((_BPW, _D), jnp.float32),
            pltpu.VMEM((_CH, 16), jnp.float32),
            pltpu.VMEM((stripe, 16), jnp.float32),
            pltpu.VMEM_SHARED((_K, 16), jnp.float32),
            pltpu.SemaphoreType.DMA,
        ],
    )
    def k(qc_hbm, idx_hbm, zq_hbm, cnt_hbm,
          idx_v, rows_v, ones_v, zero_v, shared, sem):
        cid = lax.axis_index("c")
        sid = lax.axis_index("s")
        wid = sid * 2 + cid
        pltpu.sync_copy(idx_hbm.at[pl.ds(wid * nchunk, nchunk)], idx_v)
        copies = []
        for c in range(nchunk):
            copies.append(pltpu.async_copy(
                qc_hbm.at[idx_v.at[c]],
                rows_v.at[pl.ds(c * _CH, _CH)], sem))

        col = lax.broadcasted_iota(jnp.int32, (16,), 0)
        one16 = jnp.where(col == 0, 1.0, 0.0).astype(jnp.float32)

        def ones_body(i, carry):
            ones_v[i] = one16
            return carry
        lax.fori_loop(0, _CH, ones_body, 0)

        def zero_body(i, carry):
            zero_v[i] = jnp.zeros((16,), jnp.float32)
            return carry
        lax.fori_loop(0, stripe, zero_body, 0)
        pltpu.sync_copy(zero_v, shared.at[pl.ds(sid * stripe, stripe)])
        plsc.subcore_barrier()
        for c in range(nchunk):
            pltpu.sync_copy(ones_v, shared.at[idx_v.at[c]], add=True)
        plsc.subcore_barrier()

        @pl.when(sid == 0)
        def _():
            pltpu.sync_copy(shared, cnt_hbm.at[cid])

        for cp in copies:
            cp.wait()
        pltpu.sync_copy(rows_v, zq_hbm.at[pl.ds(wid * _BPW, _BPW)])

    return k(qc, idx2d)


def _loss_body(z_ref, zq_ref, pc_ref, loss_ref, perp_ref):
    diff = zq_ref[...] - z_ref[...]
    m = jnp.sum(diff * diff) / jnp.float32(_N * _D)
    loss_ref[...] = (_BETA * m + m).reshape(1, 1)
    rows = pc_ref[pl.ds(0, _K), :] + pc_ref[pl.ds(_K, _K), :]
    counts = jnp.sum(rows, axis=1, keepdims=True)
    e_mean = counts / jnp.float32(_N)
    perp = jnp.exp(-jnp.sum(e_mean * jnp.log(e_mean + 1e-8)))
    perp_ref[...] = perp.reshape(1, 1)


def _losses(zf, zq, pcnt):
    return pl.pallas_call(
        _loss_body,
        out_shape=[
            jax.ShapeDtypeStruct((1, 1), jnp.float32),
            jax.ShapeDtypeStruct((1, 1), jnp.float32),
        ],
    )(zf, zq, pcnt)


def kernel(z, codebook, proj_w, proj_b):
    zf = z.reshape(_N, _D)
    qc, qsq = _project(codebook, proj_w, proj_b.reshape(1, _D))
    idx2d = _argmin(zf, qc, qsq).reshape(_N // _CH, _CH)
    zq, pcnt = _sc_gather_count(qc, idx2d)
    loss, perp = _losses(zf, zq, pcnt.reshape(2 * _K, 16))
    return zq.reshape(z.shape), loss[0, 0], perp[0, 0]


# speedup vs baseline: 3.3208x; 3.3208x over previous
import functools

import jax
import jax.numpy as jnp
from jax import lax
from jax.experimental import pallas as pl
from jax.experimental.pallas import tpu as pltpu
from jax.experimental.pallas import tpu_sc as plsc

_K = 8192
_D = 64
_BETA = 0.25
_N = 16384
_MT = 1024
_KC = 2048
_NW = 32
_BPW = _N // _NW
_CH = 128

_PRECISION = lax.Precision.DEFAULT


def _proj_body(cb_ref, pw_ref, pb_ref, qc_ref, qsq_ref):
    qc = lax.dot_general(
        cb_ref[...], pw_ref[...], (((1,), (1,)), ((), ())),
        precision=_PRECISION, preferred_element_type=jnp.float32,
    ) + pb_ref[...]
    qc_ref[...] = qc
    qsq_ref[...] = lax.dot_general(
        jnp.ones((1, _D), jnp.float32), qc * qc, (((1,), (1,)), ((), ())),
        precision=lax.Precision.HIGHEST, preferred_element_type=jnp.float32)


def _project(codebook, proj_w, proj_b2d):
    return pl.pallas_call(
        _proj_body,
        out_shape=[
            jax.ShapeDtypeStruct((_K, _D), jnp.float32),
            jax.ShapeDtypeStruct((1, _K), jnp.float32),
        ],
    )(codebook, proj_w, proj_b2d)


def _argmin_body(z_ref, qc_ref, qsq_ref, idx_ref):
    zt = z_ref[...]
    z_sq = jnp.sum(zt * zt, axis=1, keepdims=True)
    zt2 = -2.0 * zt
    iota = lax.broadcasted_iota(jnp.int32, (_MT, _KC), 1)
    big = jnp.int32(2 ** 30)
    bv = jnp.full((_MT, 1), jnp.inf, jnp.float32)
    bi = jnp.zeros((_MT, 1), jnp.int32)
    for c in range(_K // _KC):
        qcc = qc_ref[pl.ds(c * _KC, _KC), :]
        qs = qsq_ref[:, pl.ds(c * _KC, _KC)]
        dots2 = lax.dot_general(
            zt2, qcc, (((1,), (1,)), ((), ())),
            precision=_PRECISION, preferred_element_type=jnp.float32)
        d = (z_sq + qs) + dots2
        cmin = jnp.min(d, axis=1, keepdims=True)
        cidx = jnp.min(jnp.where(d == cmin, iota, big),
                       axis=1, keepdims=True)
        upd = cmin < bv
        bi = jnp.where(upd, cidx + (c * _KC), bi)
        bv = jnp.where(upd, cmin, bv)
    idx_ref[...] = bi


def _argmin(zf, qc, qsq):
    return pl.pallas_call(
        _argmin_body,
        grid=(_N // _MT,),
        in_specs=[
            pl.BlockSpec((_MT, _D), lambda i: (i, 0)),
            pl.BlockSpec((_K, _D), lambda i: (0, 0)),
            pl.BlockSpec((1, _K), lambda i: (0, 0)),
        ],
        out_specs=pl.BlockSpec((_MT, 1), lambda i: (i, 0)),
        out_shape=jax.ShapeDtypeStruct((_N, 1), jnp.int32),
    )(zf, qc, qsq)


def _sc_gather_count(qc, idx2d):
    mesh = plsc.VectorSubcoreMesh(core_axis_name="c", subcore_axis_name="s")
    nchunk = _BPW // _CH
    stripe = _K // 16

    @functools.partial(
        pl.kernel,
        mesh=mesh,
        compiler_params=pltpu.CompilerParams(use_tc_tiling_on_sc=False),
        out_type=[
            jax.ShapeDtypeStruct((_N, _D), jnp.float32),
            jax.ShapeDtypeStruct((2, _K, 16), jnp.float32),
        ],
        scratch_types=[
            pltpu.VMEM((nchunk, _CH), jnp.int32),
            pltpu.VMEM((_BPW, _D), jnp.float32),
            pltpu.VMEM((_CH, 16), jnp.float32),
            pltpu.VMEM((stripe, 16), jnp.float32),
            pltpu.VMEM_SHARED((_K, 16), jnp.float32),
            pltpu.SemaphoreType.DMA,
        ],
    )
    def k(qc_hbm, idx_hbm, zq_hbm, cnt_hbm,
          idx_v, rows_v, ones_v, zero_v, shared, sem):
        cid = lax.axis_index("c")
        sid = lax.axis_index("s")
        wid = sid * 2 + cid
        pltpu.sync_copy(idx_hbm.at[pl.ds(wid * nchunk, nchunk)], idx_v)
        copies = []
        for c in range(nchunk):
            copies.append(pltpu.async_copy(
                qc_hbm.at[idx_v.at[c]],
                rows_v.at[pl.ds(c * _CH, _CH)], sem))

        col = lax.broadcasted_iota(jnp.int32, (16,), 0)
        one16 = jnp.where(col == 0, 1.0, 0.0).astype(jnp.float32)

        def ones_body(i, carry):
            ones_v[i] = one16
            return carry
        lax.fori_loop(0, _CH, ones_body, 0)

        def zero_body(i, carry):
            zero_v[i] = jnp.zeros((16,), jnp.float32)
            return carry
        lax.fori_loop(0, stripe, zero_body, 0)
        pltpu.sync_copy(zero_v, shared.at[pl.ds(sid * stripe, stripe)])
        plsc.subcore_barrier()
        for c in range(nchunk):
            pltpu.sync_copy(ones_v, shared.at[idx_v.at[c]], add=True)
        plsc.subcore_barrier()

        @pl.when(sid == 0)
        def _():
            pltpu.sync_copy(shared, cnt_hbm.at[cid])

        for cp in copies:
            cp.wait()
        pltpu.sync_copy(rows_v, zq_hbm.at[pl.ds(wid * _BPW, _BPW)])

    return k(qc, idx2d)


def _loss_body(z_ref, zq_ref, pc_ref, loss_ref, perp_ref):
    diff = zq_ref[...] - z_ref[...]
    m = jnp.sum(diff * diff) / jnp.float32(_N * _D)
    loss_ref[...] = (_BETA * m + m).reshape(1, 1)
    rows = pc_ref[pl.ds(0, _K), :] + pc_ref[pl.ds(_K, _K), :]
    counts = jnp.sum(rows, axis=1, keepdims=True)
    e_mean = counts / jnp.float32(_N)
    perp = jnp.exp(-jnp.sum(e_mean * jnp.log(e_mean + 1e-8)))
    perp_ref[...] = perp.reshape(1, 1)


def _losses(zf, zq, pcnt):
    return pl.pallas_call(
        _loss_body,
        out_shape=[
            jax.ShapeDtypeStruct((1, 1), jnp.float32),
            jax.ShapeDtypeStruct((1, 1), jnp.float32),
        ],
    )(zf, zq, pcnt)


def kernel(z, codebook, proj_w, proj_b):
    zf = z.reshape(_N, _D)
    qc, qsq = _project(codebook, proj_w, proj_b.reshape(1, _D))
    idx2d = _argmin(zf, qc, qsq).reshape(_N // _CH, _CH)
    zq, pcnt = _sc_gather_count(qc, idx2d)
    loss, perp = _losses(zf, zq, pcnt.reshape(2 * _K, 16))
    return zq.reshape(z.shape), loss[0, 0], perp[0, 0]
